# Initial kernel scaffold; baseline (speedup 1.0000x reference)
import jax
import jax.numpy as jnp
from jax.experimental import pallas as pl


def kernel(x, edge_index, adj, s, W_rel1, b_rel1, W_root1, b_root1, W_rel2, b_rel2, W_root2, b_root2, W_rel3, b_rel3, W_root3, b_root3):
    raise NotImplementedError("write your pallas kernel here")


# speedup vs baseline: 4.3073x; 4.3073x over previous
import functools

import jax
import jax.numpy as jnp
from jax import lax
from jax.experimental import pallas as pl
from jax.experimental.pallas import tpu as pltpu
from jax.experimental.pallas import tpu_sc as plsc

_N = 68
_E = 544
_H = 128
_C = 8
_LANES = 16
_NSEG = _N * _N


def _sc_build_adj(src, dst):
    mesh = plsc.VectorSubcoreMesh(core_axis_name="c", subcore_axis_name="s")

    @functools.partial(
        pl.kernel,
        out_type=jax.ShapeDtypeStruct((_NSEG,), jnp.float32),
        mesh=mesh,
        scratch_types=[
            pltpu.VMEM((_E,), jnp.int32),
            pltpu.VMEM((_E,), jnp.int32),
            pltpu.VMEM((_NSEG,), jnp.float32),
        ],
        compiler_params=pltpu.CompilerParams(needs_layout_passes=False),
    )
    def build(src_hbm, dst_hbm, out_hbm, src_v, dst_v, acc_v):
        cid = lax.axis_index("c")
        sid = lax.axis_index("s")

        @pl.when(jnp.logical_and(cid == 0, sid == 0))
        def _():
            pltpu.sync_copy(src_hbm, src_v)
            pltpu.sync_copy(dst_hbm, dst_v)

            def zero_body(i, carry):
                acc_v[pl.ds(i * _LANES, _LANES)] = jnp.zeros(
                    (_LANES,), jnp.float32)
                return carry

            lax.fori_loop(0, _NSEG // _LANES, zero_body, 0)

            ones = jnp.ones((_LANES,), jnp.float32)
            lane = lax.iota(jnp.int32, _LANES)

            def edge_body(i, carry):
                s_ids = src_v[pl.ds(i * _LANES, _LANES)]
                d_ids = dst_v[pl.ds(i * _LANES, _LANES)]
                flat = d_ids * _N + s_ids
                for l in range(_LANES):
                    plsc.addupdate_scatter(
                        acc_v, [flat], ones, mask=lane == l)
                return carry

            lax.fori_loop(0, _E // _LANES, edge_body, 0)
            pltpu.sync_copy(acc_v, out_hbm)

    return build(src, dst)


def _tc_body(a_ref, x_ref, st_ref,
             wr1_ref, wo1_ref, b1_ref,
             wr2_ref, wo2_ref, b2_ref,
             wr3_ref, wo3_ref, b3_ref,
             out_ref):
    def dot(p, q):
        return lax.dot_general(p, q, (((1,), (0,)), ((), ())),
                               preferred_element_type=jnp.float32)

    a = a_ref[...]
    h = x_ref[...]

    for wr, wo, b in ((wr1_ref, wo1_ref, b1_ref),
                      (wr2_ref, wo2_ref, b2_ref),
                      (wr3_ref, wo3_ref, b3_ref)):
        h = jnp.maximum(dot(a, dot(h, wr[...])) + dot(h, wo[...]) + b[...],
                        0.0)

    st = st_ref[...]
    m = jnp.max(st, axis=0, keepdims=True)
    e = jnp.exp(st - m)
    sst = e / jnp.sum(e, axis=0, keepdims=True)
    out_ref[...] = dot(sst, h)


def kernel(x, edge_index, adj, s,
           W_rel1, b_rel1, W_root1, b_root1,
           W_rel2, b_rel2, W_root2, b_root2,
           W_rel3, b_rel3, W_root3, b_root3):
    del adj
    src = edge_index[0].astype(jnp.int32)
    dst = edge_index[1].astype(jnp.int32)

    a = _sc_build_adj(src, dst).reshape(_N, _N)
    st = s[0].T

    out = pl.pallas_call(
        _tc_body,
        out_shape=jax.ShapeDtypeStruct((_C, _H), jnp.float32),
    )(a, x, st,
      W_rel1.T, W_root1.T, (b_rel1 + b_root1).reshape(1, _H),
      W_rel2.T, W_root2.T, (b_rel2 + b_root2).reshape(1, _H),
      W_rel3.T, W_root3.T, (b_rel3 + b_root3).reshape(1, _H))

    return out.reshape(1, _C, _H)


# speedup vs baseline: 4.7225x; 1.0964x over previous
import functools

import jax
import jax.numpy as jnp
from jax import lax
from jax.experimental import pallas as pl
from jax.experimental.pallas import tpu as pltpu
from jax.experimental.pallas import tpu_sc as plsc

_N = 68
_E = 544
_H = 128
_C = 8
_LANES = 16


def _sc_build_adj(edge_index):
    mesh = plsc.VectorSubcoreMesh(core_axis_name="c", subcore_axis_name="s")

    @functools.partial(
        pl.kernel,
        out_type=jax.ShapeDtypeStruct((_N, _N), jnp.float32),
        mesh=mesh,
        scratch_types=[
            pltpu.VMEM((_E,), jnp.int32),
            pltpu.VMEM((_E,), jnp.int32),
            pltpu.VMEM((_N, _N), jnp.float32),
        ],
        compiler_params=pltpu.CompilerParams(needs_layout_passes=False),
    )
    def build(ei_hbm, out_hbm, src_v, dst_v, acc_v):
        cid = lax.axis_index("c")
        sid = lax.axis_index("s")

        @pl.when(jnp.logical_and(cid == 0, sid == 0))
        def _():
            pltpu.sync_copy(ei_hbm.at[0], src_v)
            pltpu.sync_copy(ei_hbm.at[1], dst_v)

            zeros = jnp.zeros((_LANES,), jnp.float32)

            def zero_body(i, carry):
                acc_v[i, pl.ds(0, _LANES)] = zeros
                acc_v[i, pl.ds(16, _LANES)] = zeros
                acc_v[i, pl.ds(32, _LANES)] = zeros
                acc_v[i, pl.ds(48, _LANES)] = zeros
                acc_v[i, pl.ds(_N - _LANES, _LANES)] = zeros
                return carry

            lax.fori_loop(0, _N, zero_body, 0)

            ones = jnp.ones((_LANES,), jnp.float32)
            lane = lax.iota(jnp.int32, _LANES)

            def edge_body(i, carry):
                s_ids = src_v[pl.ds(i * _LANES, _LANES)]
                d_ids = dst_v[pl.ds(i * _LANES, _LANES)]
                for l in range(_LANES):
                    plsc.addupdate_scatter(
                        acc_v, [d_ids, s_ids], ones, mask=lane == l)
                return carry

            lax.fori_loop(0, _E // _LANES, edge_body, 0)
            pltpu.sync_copy(acc_v, out_hbm)

    return build(edge_index)


def _tc_body(a_ref, x_ref, s_ref,
             wr1_ref, br1_ref, wo1_ref, bo1_ref,
             wr2_ref, br2_ref, wo2_ref, bo2_ref,
             wr3_ref, br3_ref, wo3_ref, bo3_ref,
             out_ref):
    def dot(p, q):
        return lax.dot_general(p, q, (((1,), (0,)), ((), ())),
                               preferred_element_type=jnp.float32)

    def dot_rt(p, q):
        return lax.dot_general(p, q, (((1,), (1,)), ((), ())),
                               preferred_element_type=jnp.float32)

    a = a_ref[...]
    h = x_ref[...]

    for wr, br, wo, bo in ((wr1_ref, br1_ref, wo1_ref, bo1_ref),
                           (wr2_ref, br2_ref, wo2_ref, bo2_ref),
                           (wr3_ref, br3_ref, wo3_ref, bo3_ref)):
        h = jnp.maximum(
            dot(a, dot_rt(h, wr[...])) + dot_rt(h, wo[...])
            + (br[...] + bo[...])[None, :],
            0.0)

    st = s_ref[0]
    m = jnp.max(st, axis=1, keepdims=True)
    e = jnp.exp(st - m)
    ss = e / jnp.sum(e, axis=1, keepdims=True)
    out_ref[...] = lax.dot_general(ss, h, (((0,), (0,)), ((), ())),
                                   preferred_element_type=jnp.float32)


def kernel(x, edge_index, adj, s,
           W_rel1, b_rel1, W_root1, b_root1,
           W_rel2, b_rel2, W_root2, b_root2,
           W_rel3, b_rel3, W_root3, b_root3):
    del adj
    a = _sc_build_adj(edge_index.astype(jnp.int32))

    out = pl.pallas_call(
        _tc_body,
        out_shape=jax.ShapeDtypeStruct((_C, _H), jnp.float32),
    )(a, x, s,
      W_rel1, b_rel1, W_root1, b_root1,
      W_rel2, b_rel2, W_root2, b_root2,
      W_rel3, b_rel3, W_root3, b_root3)

    return out.reshape(1, _C, _H)
